# baseline (speedup 1.0000x reference)
import functools

import jax
import jax.numpy as jnp
from jax.experimental import pallas as pl
from jax.experimental.pallas import tpu as pltpu

E = 8
H = 1024
I = 4096
K = 2
T = 2048
CAP = 1024
P = T * K
BUF_ROWS = E * CAP + 8

BI = 512
NI = I // BI
BR = 256


def _dense_body(counts_ref, buf_ref, w1_ref, w3_ref, w2_ref, out_ref, acc_ref):
    e = pl.program_id(0)
    i = pl.program_id(1)
    cnt = counts_ref[e]
    nrb = (cnt + BR - 1) // BR

    @pl.when(i == 0)
    def _():
        acc_ref[...] = jnp.zeros_like(acc_ref)

    def body(rb, carry):
        rows = buf_ref[pl.ds(rb * BR, BR), :].astype(jnp.bfloat16)
        gate = jnp.dot(rows, w1_ref[0], preferred_element_type=jnp.float32)
        up = jnp.dot(rows, w3_ref[0], preferred_element_type=jnp.float32)
        act = (gate * jax.nn.sigmoid(gate) * up).astype(jnp.bfloat16)
        partial = jnp.dot(act, w2_ref[0], preferred_element_type=jnp.float32)
        acc_ref[pl.ds(rb * BR, BR), :] += partial
        return carry

    jax.lax.fori_loop(0, nrb, body, 0)

    @pl.when(i == NI - 1)
    def _():
        out_ref[...] = acc_ref[...]


def _dense(buf, counts, w1b, w3b, w2b):
    return pl.pallas_call(
        _dense_body,
        grid=(E, NI),
        in_specs=[
            pl.BlockSpec(memory_space=pltpu.SMEM),
            pl.BlockSpec((CAP, H), lambda e, i: (e, 0)),
            pl.BlockSpec((1, H, BI), lambda e, i: (e, 0, i)),
            pl.BlockSpec((1, H, BI), lambda e, i: (e, 0, i)),
            pl.BlockSpec((1, BI, H), lambda e, i: (e, i, 0)),
        ],
        out_specs=pl.BlockSpec((CAP, H), lambda e, i: (e, 0)),
        out_shape=jax.ShapeDtypeStruct((E * CAP, H), jnp.float32),
        scratch_shapes=[pltpu.VMEM((CAP, H), jnp.float32)],
        compiler_params=pltpu.CompilerParams(
            dimension_semantics=("arbitrary", "arbitrary"),
        ),
    )(counts, buf, w1b, w3b, w2b)


BT = 128


def _combine_body(g_ref, s_ref, o_ref):
    gs = g_ref[...] * s_ref[...][:, 0:1]
    o_ref[...] = gs.reshape(BT, K, H).sum(axis=1)


def _combine(gathered, scale2d):
    return pl.pallas_call(
        _combine_body,
        grid=(T // BT,),
        in_specs=[
            pl.BlockSpec((BT * K, H), lambda t: (t, 0)),
            pl.BlockSpec((BT * K, 128), lambda t: (t, 0)),
        ],
        out_specs=pl.BlockSpec((BT, H), lambda t: (t, 0)),
        out_shape=jax.ShapeDtypeStruct((T, H), jnp.float32),
    )(gathered, scale2d)


def _dispatch_host(idx_flat, vals_flat, x):
    oh = jax.nn.one_hot(idx_flat, E, dtype=jnp.int32)
    pos = ((jnp.cumsum(oh, axis=0) - 1) * oh).sum(axis=1)
    counts = oh.sum(axis=0)
    keep = pos < CAP
    dst = idx_flat * CAP + pos
    dst_scatter = jnp.where(keep, dst, E * CAP)
    combine_idx = jnp.where(keep, dst, idx_flat * CAP + CAP - 1)
    scale = jnp.where(keep, vals_flat, 0.0)
    x_rep = jnp.repeat(x, K, axis=0)
    buf = jnp.zeros((BUF_ROWS, H), jnp.float32).at[dst_scatter].set(x_rep)
    counts16 = jnp.zeros((16,), jnp.int32).at[:E].set(counts)
    return buf, counts16, combine_idx, scale


def kernel(hidden_states, top_k_indices, top_k_values, w1, w2, w3):
    idx_flat = top_k_indices.reshape(-1)
    vals_flat = top_k_values.reshape(-1)
    buf, counts16, combine_idx, scale = _dispatch_host(
        idx_flat, vals_flat, hidden_states)
    w1b = w1.astype(jnp.bfloat16)
    w3b = w3.astype(jnp.bfloat16)
    w2b = w2.astype(jnp.bfloat16)
    out_buf = _dense(buf, counts16, w1b, w3b, w2b)
    gathered = out_buf[combine_idx]
    scale2d = jnp.broadcast_to(scale[:, None], (P, 128))
    return _combine(gathered, scale2d)


# speedup vs baseline: 1.1142x; 1.1142x over previous
import functools

import jax
import jax.numpy as jnp
from jax import lax
from jax.experimental import pallas as pl
from jax.experimental.pallas import tpu as pltpu
from jax.experimental.pallas import tpu_sc as plsc

E = 8
H = 1024
I = 4096
K = 2
T = 2048
CAP = 1024
P = T * K
TRASH = E * CAP
BUF_ROWS = E * CAP + 8

BI = 512
NI = I // BI
BR = 256

NC = 2
NS = 16
L = 16
NW = NC * NS
CHUNK = P // NW
NVEC = CHUNK // L
SUB = 32
NSUB = CHUNK // SUB

_MESH = plsc.VectorSubcoreMesh(core_axis_name="c", subcore_axis_name="s")


def _iota16():
    return lax.iota(jnp.int32, L)


def _ones(m):
    return jnp.where(m, 1, 0)


def _psum16(c):
    lane = _iota16()
    for k in (1, 2, 4, 8):
        sh = c[jnp.maximum(lane - k, 0)]
        c = c + jnp.where(lane >= k, sh, 0)
    return c


def _splat_last(c):
    return c[jnp.full((L,), L - 1, jnp.int32)]


@functools.partial(
    pl.kernel,
    out_type=(
        jax.ShapeDtypeStruct((BUF_ROWS, H), jnp.float32),
        jax.ShapeDtypeStruct((16,), jnp.int32),
        jax.ShapeDtypeStruct((P,), jnp.int32),
        jax.ShapeDtypeStruct((P,), jnp.float32),
    ),
    mesh=_MESH,
    scratch_types=(
        pltpu.VMEM((P,), jnp.int32),
        pltpu.VMEM((CHUNK,), jnp.float32),
        pltpu.VMEM((CHUNK,), jnp.int32),
        pltpu.VMEM((CHUNK,), jnp.float32),
        pltpu.VMEM((16,), jnp.int32),
        pltpu.VMEM((SUB,), jnp.int32),
        pltpu.VMEM((SUB,), jnp.int32),
        pltpu.VMEM((SUB,), jnp.int32),
        pltpu.VMEM((SUB,), jnp.int32),
        pltpu.VMEM((SUB, H), jnp.float32),
        pltpu.VMEM((SUB, H), jnp.float32),
        pltpu.SemaphoreType.DMA,
        pltpu.SemaphoreType.DMA,
    ),
)
def _dispatch_sc(idx_hbm, vals_hbm, x_hbm,
                 buf_hbm, counts_hbm, cidx_hbm, scale_hbm,
                 idx_v, vals_v, cidx_v, scale_v, cnt_v,
                 tok0_v, dst0_v, tok1_v, dst1_v, rows0_v, rows1_v,
                 gsem, ssem):
    wid = lax.axis_index("s") * NC + lax.axis_index("c")
    base = wid * CHUNK

    pltpu.sync_copy(idx_hbm, idx_v)
    pltpu.sync_copy(vals_hbm.at[pl.ds(base, CHUNK)], vals_v)

    zeros = jnp.zeros((L,), jnp.int32)
    lanei = _iota16()

    def prefix_body(i, cs):
        v = idx_v[pl.ds(i * L, L)]
        return tuple(cs[e] + _ones(v == e) for e in range(E))

    carry_vecs = lax.fori_loop(0, wid * NVEC, prefix_body,
                               tuple(zeros for _ in range(E)))
    carries = [_splat_last(_psum16(carry_vecs[e])) for e in range(E)]

    for j in range(NVEC):
        v = idx_v[pl.ds(base + j * L, L)]
        pos = zeros
        for e in range(E):
            m = v == e
            csum = _psum16(_ones(m))
            pos = jnp.where(m, carries[e] + csum - 1, pos)
            carries[e] = carries[e] + _splat_last(csum)

        vals = vals_v[pl.ds(j * L, L)]
        keep = pos < CAP
        dst = v * CAP + pos
        cidx_v[pl.ds(j * L, L)] = jnp.where(keep, dst, v * CAP + (CAP - 1))
        scale_v[pl.ds(j * L, L)] = jnp.where(keep, vals, 0.0)
        dstv = jnp.where(keep, dst, TRASH)
        tokv = lax.shift_right_logical(
            jnp.full((L,), base + j * L, jnp.int32) + lanei, 1)
        s, q = divmod(j, NVEC // NSUB)
        tref = tok0_v if s % 2 == 0 else tok1_v
        dref = dst0_v if s % 2 == 0 else dst1_v
        tref[pl.ds(q * L, L)] = tokv
        dref[pl.ds(q * L, L)] = dstv
        if q == NVEC // NSUB - 1:
            rref = rows0_v if s % 2 == 0 else rows1_v
            pltpu.async_copy(x_hbm.at[tref], rref, gsem).wait()
            pltpu.async_copy(rref, buf_hbm.at[dref], ssem).wait()

    @pl.when(wid == NW - 1)
    def _():
        tot = zeros
        for e in range(E):
            tot = jnp.where(lanei == e, carries[e], tot)
        cnt_v[...] = tot
        pltpu.sync_copy(cnt_v, counts_hbm)

    pltpu.sync_copy(cidx_v, cidx_hbm.at[pl.ds(base, CHUNK)])
    pltpu.sync_copy(scale_v, scale_hbm.at[pl.ds(base, CHUNK)])


@functools.partial(
    pl.kernel,
    out_type=jax.ShapeDtypeStruct((P, H), jnp.float32),
    mesh=_MESH,
    scratch_types=(
        pltpu.VMEM((CHUNK,), jnp.int32),
        pltpu.VMEM((SUB,), jnp.int32),
        pltpu.VMEM((SUB,), jnp.int32),
        pltpu.VMEM((SUB, H), jnp.float32),
        pltpu.VMEM((SUB, H), jnp.float32),
        pltpu.SemaphoreType.DMA,
    ),
)
def _gather_sc(outbuf_hbm, cidx_hbm, gathered_hbm,
               cidx_v, sidx0_v, sidx1_v, rows0_v, rows1_v, gsem):
    wid = lax.axis_index("s") * NC + lax.axis_index("c")
    base = wid * CHUNK
    pltpu.sync_copy(cidx_hbm.at[pl.ds(base, CHUNK)], cidx_v)
    for s in range(NSUB):
        sref = sidx0_v if s % 2 == 0 else sidx1_v
        rref = rows0_v if s % 2 == 0 else rows1_v
        for q in range(SUB // L):
            sref[pl.ds(q * L, L)] = cidx_v[pl.ds(s * SUB + q * L, L)]
        pltpu.async_copy(outbuf_hbm.at[sref], rref, gsem).wait()
        pltpu.sync_copy(rref, gathered_hbm.at[pl.ds(base + s * SUB, SUB)])


def _dense_body(counts_ref, buf_ref, w1_ref, w3_ref, w2_ref, out_ref, acc_ref):
    e = pl.program_id(0)
    i = pl.program_id(1)
    cnt = counts_ref[e]
    nrb = (cnt + BR - 1) // BR

    @pl.when(i == 0)
    def _():
        acc_ref[...] = jnp.zeros_like(acc_ref)

    def body(rb, carry):
        rows = buf_ref[pl.ds(rb * BR, BR), :].astype(jnp.bfloat16)
        gate = jnp.dot(rows, w1_ref[0], preferred_element_type=jnp.float32)
        up = jnp.dot(rows, w3_ref[0], preferred_element_type=jnp.float32)
        act = (gate * jax.nn.sigmoid(gate) * up).astype(jnp.bfloat16)
        partial = jnp.dot(act, w2_ref[0], preferred_element_type=jnp.float32)
        acc_ref[pl.ds(rb * BR, BR), :] += partial
        return carry

    lax.fori_loop(0, nrb, body, 0)

    @pl.when(i == NI - 1)
    def _():
        out_ref[...] = acc_ref[...]


def _dense(buf, counts, w1b, w3b, w2b):
    return pl.pallas_call(
        _dense_body,
        grid=(E, NI),
        in_specs=[
            pl.BlockSpec(memory_space=pltpu.SMEM),
            pl.BlockSpec((CAP, H), lambda e, i: (e, 0)),
            pl.BlockSpec((1, H, BI), lambda e, i: (e, 0, i)),
            pl.BlockSpec((1, H, BI), lambda e, i: (e, 0, i)),
            pl.BlockSpec((1, BI, H), lambda e, i: (e, i, 0)),
        ],
        out_specs=pl.BlockSpec((CAP, H), lambda e, i: (e, 0)),
        out_shape=jax.ShapeDtypeStruct((E * CAP, H), jnp.float32),
        scratch_shapes=[pltpu.VMEM((CAP, H), jnp.float32)],
        compiler_params=pltpu.CompilerParams(
            dimension_semantics=("arbitrary", "arbitrary"),
        ),
    )(counts, buf, w1b, w3b, w2b)


BT = 128


def _combine_body(g_ref, s_ref, o_ref):
    gs = g_ref[...] * s_ref[...][:, 0:1]
    o_ref[...] = gs.reshape(BT, K, H).sum(axis=1)


def _combine(gathered, scale2d):
    return pl.pallas_call(
        _combine_body,
        grid=(T // BT,),
        in_specs=[
            pl.BlockSpec((BT * K, H), lambda t: (t, 0)),
            pl.BlockSpec((BT * K, 128), lambda t: (t, 0)),
        ],
        out_specs=pl.BlockSpec((BT, H), lambda t: (t, 0)),
        out_shape=jax.ShapeDtypeStruct((T, H), jnp.float32),
    )(gathered, scale2d)


def kernel(hidden_states, top_k_indices, top_k_values, w1, w2, w3):
    idx_flat = top_k_indices.reshape(-1)
    vals_flat = top_k_values.reshape(-1)
    buf, counts16, cidx, scale = _dispatch_sc(
        idx_flat, vals_flat, hidden_states)
    out_buf = _dense(buf, counts16, w1.astype(jnp.bfloat16),
                     w3.astype(jnp.bfloat16), w2.astype(jnp.bfloat16))
    gathered = _gather_sc(out_buf, cidx)
    scale2d = jnp.broadcast_to(scale[:, None], (P, 128))
    return _combine(gathered, scale2d)


# speedup vs baseline: 1.2032x; 1.0798x over previous
import functools

import jax
import jax.numpy as jnp
from jax import lax
from jax.experimental import pallas as pl
from jax.experimental.pallas import tpu as pltpu
from jax.experimental.pallas import tpu_sc as plsc

E = 8
H = 1024
I = 4096
K = 2
T = 2048
CAP = 1024
P = T * K
TRASH = E * CAP
BUF_ROWS = E * CAP + 8

BI = 1024
NI = I // BI
BR = 256

NC = 2
NS = 16
L = 16
NW = NC * NS
CHUNK = P // NW
NVEC = CHUNK // L
SUB = 32
NSUB = CHUNK // SUB

_MESH = plsc.VectorSubcoreMesh(core_axis_name="c", subcore_axis_name="s")


def _iota16():
    return lax.iota(jnp.int32, L)


def _ones(m):
    return jnp.where(m, 1, 0)


def _psum16(c):
    lane = _iota16()
    for k in (1, 2, 4, 8):
        sh = c[jnp.maximum(lane - k, 0)]
        c = c + jnp.where(lane >= k, sh, 0)
    return c


def _splat_last(c):
    return c[jnp.full((L,), L - 1, jnp.int32)]


@functools.partial(
    pl.kernel,
    out_type=(
        jax.ShapeDtypeStruct((BUF_ROWS, H), jnp.float32),
        jax.ShapeDtypeStruct((16,), jnp.int32),
        jax.ShapeDtypeStruct((P,), jnp.int32),
        jax.ShapeDtypeStruct((P,), jnp.float32),
    ),
    mesh=_MESH,
    scratch_types=(
        pltpu.VMEM((P,), jnp.int32),
        pltpu.VMEM((CHUNK,), jnp.float32),
        pltpu.VMEM((CHUNK,), jnp.int32),
        pltpu.VMEM((CHUNK,), jnp.float32),
        pltpu.VMEM((16,), jnp.int32),
        pltpu.VMEM((SUB,), jnp.int32),
        pltpu.VMEM((SUB,), jnp.int32),
        pltpu.VMEM((SUB,), jnp.int32),
        pltpu.VMEM((SUB,), jnp.int32),
        pltpu.VMEM((SUB, H), jnp.float32),
        pltpu.VMEM((SUB, H), jnp.float32),
        pltpu.SemaphoreType.DMA,
        pltpu.SemaphoreType.DMA,
    ),
)
def _dispatch_sc(idx_hbm, vals_hbm, x_hbm,
                 buf_hbm, counts_hbm, cidx_hbm, scale_hbm,
                 idx_v, vals_v, cidx_v, scale_v, cnt_v,
                 tok0_v, dst0_v, tok1_v, dst1_v, rows0_v, rows1_v,
                 gsem, ssem):
    wid = lax.axis_index("s") * NC + lax.axis_index("c")
    base = wid * CHUNK

    pltpu.sync_copy(idx_hbm, idx_v)
    pltpu.sync_copy(vals_hbm.at[pl.ds(base, CHUNK)], vals_v)

    zeros = jnp.zeros((L,), jnp.int32)
    lanei = _iota16()

    def prefix_body(i, cs):
        v = idx_v[pl.ds(i * L, L)]
        return tuple(cs[e] + _ones(v == e) for e in range(E))

    carry_vecs = lax.fori_loop(0, wid * NVEC, prefix_body,
                               tuple(zeros for _ in range(E)))
    carries = [_splat_last(_psum16(carry_vecs[e])) for e in range(E)]

    for j in range(NVEC):
        v = idx_v[pl.ds(base + j * L, L)]
        pos = zeros
        for e in range(E):
            m = v == e
            csum = _psum16(_ones(m))
            pos = jnp.where(m, carries[e] + csum - 1, pos)
            carries[e] = carries[e] + _splat_last(csum)

        vals = vals_v[pl.ds(j * L, L)]
        keep = pos < CAP
        dst = v * CAP + pos
        cidx_v[pl.ds(j * L, L)] = jnp.where(keep, dst, v * CAP + (CAP - 1))
        scale_v[pl.ds(j * L, L)] = jnp.where(keep, vals, 0.0)
        dstv = jnp.where(keep, dst, TRASH)
        tokv = lax.shift_right_logical(
            jnp.full((L,), base + j * L, jnp.int32) + lanei, 1)
        s, q = divmod(j, NVEC // NSUB)
        tref = tok0_v if s % 2 == 0 else tok1_v
        dref = dst0_v if s % 2 == 0 else dst1_v
        tref[pl.ds(q * L, L)] = tokv
        dref[pl.ds(q * L, L)] = dstv
        if q == NVEC // NSUB - 1:
            rref = rows0_v if s % 2 == 0 else rows1_v
            pltpu.async_copy(x_hbm.at[tref], rref, gsem).wait()
            pltpu.async_copy(rref, buf_hbm.at[dref], ssem).wait()

    @pl.when(wid == NW - 1)
    def _():
        tot = zeros
        for e in range(E):
            tot = jnp.where(lanei == e, carries[e], tot)
        cnt_v[...] = tot
        pltpu.sync_copy(cnt_v, counts_hbm)

    pltpu.sync_copy(cidx_v, cidx_hbm.at[pl.ds(base, CHUNK)])
    pltpu.sync_copy(scale_v, scale_hbm.at[pl.ds(base, CHUNK)])


@functools.partial(
    pl.kernel,
    out_type=jax.ShapeDtypeStruct((P, H), jnp.float32),
    mesh=_MESH,
    scratch_types=(
        pltpu.VMEM((CHUNK,), jnp.int32),
        pltpu.VMEM((SUB,), jnp.int32),
        pltpu.VMEM((SUB,), jnp.int32),
        pltpu.VMEM((SUB, H), jnp.float32),
        pltpu.VMEM((SUB, H), jnp.float32),
        pltpu.SemaphoreType.DMA,
    ),
)
def _gather_sc(outbuf_hbm, cidx_hbm, gathered_hbm,
               cidx_v, sidx0_v, sidx1_v, rows0_v, rows1_v, gsem):
    wid = lax.axis_index("s") * NC + lax.axis_index("c")
    base = wid * CHUNK
    pltpu.sync_copy(cidx_hbm.at[pl.ds(base, CHUNK)], cidx_v)
    for s in range(NSUB):
        sref = sidx0_v if s % 2 == 0 else sidx1_v
        rref = rows0_v if s % 2 == 0 else rows1_v
        for q in range(SUB // L):
            sref[pl.ds(q * L, L)] = cidx_v[pl.ds(s * SUB + q * L, L)]
        pltpu.async_copy(outbuf_hbm.at[sref], rref, gsem).wait()
        pltpu.sync_copy(rref, gathered_hbm.at[pl.ds(base + s * SUB, SUB)])


def _dense_body(counts_ref, buf_ref, w1_ref, w3_ref, w2_ref, out_ref, acc_ref):
    e = pl.program_id(0)
    i = pl.program_id(1)
    cnt = counts_ref[e]
    nrb = (cnt + BR - 1) // BR

    @pl.when(i == 0)
    def _():
        acc_ref[...] = jnp.zeros_like(acc_ref)

    def body(rb, carry):
        rows = buf_ref[pl.ds(rb * BR, BR), :].astype(jnp.bfloat16)
        gate = jnp.dot(rows, w1_ref[0], preferred_element_type=jnp.float32)
        up = jnp.dot(rows, w3_ref[0], preferred_element_type=jnp.float32)
        act = (gate * jax.nn.sigmoid(gate) * up).astype(jnp.bfloat16)
        partial = jnp.dot(act, w2_ref[0], preferred_element_type=jnp.float32)
        acc_ref[pl.ds(rb * BR, BR), :] += partial
        return carry

    lax.fori_loop(0, nrb, body, 0)

    @pl.when(i == NI - 1)
    def _():
        out_ref[...] = acc_ref[...]


def _dense(buf, counts, w1b, w3b, w2b):
    return pl.pallas_call(
        _dense_body,
        grid=(E, NI),
        in_specs=[
            pl.BlockSpec(memory_space=pltpu.SMEM),
            pl.BlockSpec((CAP, H), lambda e, i: (e, 0)),
            pl.BlockSpec((1, H, BI), lambda e, i: (e, 0, i)),
            pl.BlockSpec((1, H, BI), lambda e, i: (e, 0, i)),
            pl.BlockSpec((1, BI, H), lambda e, i: (e, i, 0)),
        ],
        out_specs=pl.BlockSpec((CAP, H), lambda e, i: (e, 0)),
        out_shape=jax.ShapeDtypeStruct((E * CAP, H), jnp.float32),
        scratch_shapes=[pltpu.VMEM((CAP, H), jnp.float32)],
        compiler_params=pltpu.CompilerParams(
            dimension_semantics=("arbitrary", "arbitrary"),
        ),
    )(counts, buf, w1b, w3b, w2b)


BT = 128


def _combine_body(g_ref, s_ref, o_ref):
    gs = g_ref[...] * s_ref[...][:, 0:1]
    o_ref[...] = gs.reshape(BT, K, H).sum(axis=1)


def _combine(gathered, scale2d):
    return pl.pallas_call(
        _combine_body,
        grid=(T // BT,),
        in_specs=[
            pl.BlockSpec((BT * K, H), lambda t: (t, 0)),
            pl.BlockSpec((BT * K, 128), lambda t: (t, 0)),
        ],
        out_specs=pl.BlockSpec((BT, H), lambda t: (t, 0)),
        out_shape=jax.ShapeDtypeStruct((T, H), jnp.float32),
    )(gathered, scale2d)


def kernel(hidden_states, top_k_indices, top_k_values, w1, w2, w3):
    idx_flat = top_k_indices.reshape(-1)
    vals_flat = top_k_values.reshape(-1)
    buf, counts16, cidx, scale = _dispatch_sc(
        idx_flat, vals_flat, hidden_states)
    out_buf = _dense(buf, counts16, w1.astype(jnp.bfloat16),
                     w3.astype(jnp.bfloat16), w2.astype(jnp.bfloat16))
    gathered = _gather_sc(out_buf, cidx)
    scale2d = jnp.broadcast_to(scale[:, None], (P, 128))
    return _combine(gathered, scale2d)


# speedup vs baseline: 1.2338x; 1.0255x over previous
import functools

import jax
import jax.numpy as jnp
from jax import lax
from jax.experimental import pallas as pl
from jax.experimental.pallas import tpu as pltpu
from jax.experimental.pallas import tpu_sc as plsc

E = 8
H = 1024
I = 4096
K = 2
T = 2048
CAP = 1024
P = T * K
TRASH = E * CAP
BUF_ROWS = E * CAP + 8

BI = 2048
NI = I // BI
BR = 256

NC = 2
NS = 16
L = 16
NW = NC * NS
CHUNK = P // NW
NVEC = CHUNK // L
SUB = 32
NSUB = CHUNK // SUB

_MESH = plsc.VectorSubcoreMesh(core_axis_name="c", subcore_axis_name="s")


def _iota16():
    return lax.iota(jnp.int32, L)


def _ones(m):
    return jnp.where(m, 1, 0)


def _psum16(c):
    lane = _iota16()
    for k in (1, 2, 4, 8):
        sh = c[jnp.maximum(lane - k, 0)]
        c = c + jnp.where(lane >= k, sh, 0)
    return c


def _splat_last(c):
    return c[jnp.full((L,), L - 1, jnp.int32)]


@functools.partial(
    pl.kernel,
    out_type=(
        jax.ShapeDtypeStruct((BUF_ROWS, H), jnp.float32),
        jax.ShapeDtypeStruct((16,), jnp.int32),
        jax.ShapeDtypeStruct((P,), jnp.int32),
        jax.ShapeDtypeStruct((P,), jnp.float32),
    ),
    mesh=_MESH,
    scratch_types=(
        pltpu.VMEM((P,), jnp.int32),
        pltpu.VMEM((CHUNK,), jnp.float32),
        pltpu.VMEM((CHUNK,), jnp.int32),
        pltpu.VMEM((CHUNK,), jnp.float32),
        pltpu.VMEM((16,), jnp.int32),
        pltpu.VMEM((SUB,), jnp.int32),
        pltpu.VMEM((SUB,), jnp.int32),
        pltpu.VMEM((SUB,), jnp.int32),
        pltpu.VMEM((SUB,), jnp.int32),
        pltpu.VMEM((SUB, H), jnp.float32),
        pltpu.VMEM((SUB, H), jnp.float32),
        pltpu.SemaphoreType.DMA,
        pltpu.SemaphoreType.DMA,
    ),
)
def _dispatch_sc(idx_hbm, vals_hbm, x_hbm,
                 buf_hbm, counts_hbm, cidx_hbm, scale_hbm,
                 idx_v, vals_v, cidx_v, scale_v, cnt_v,
                 tok0_v, dst0_v, tok1_v, dst1_v, rows0_v, rows1_v,
                 gsem, ssem):
    wid = lax.axis_index("s") * NC + lax.axis_index("c")
    base = wid * CHUNK

    pltpu.sync_copy(idx_hbm, idx_v)
    pltpu.sync_copy(vals_hbm.at[pl.ds(base, CHUNK)], vals_v)

    zeros = jnp.zeros((L,), jnp.int32)
    lanei = _iota16()

    def prefix_body(i, cs):
        v = idx_v[pl.ds(i * L, L)]
        return tuple(cs[e] + _ones(v == e) for e in range(E))

    carry_vecs = lax.fori_loop(0, wid * NVEC, prefix_body,
                               tuple(zeros for _ in range(E)))
    carries = [_splat_last(_psum16(carry_vecs[e])) for e in range(E)]

    for j in range(NVEC):
        v = idx_v[pl.ds(base + j * L, L)]
        pos = zeros
        for e in range(E):
            m = v == e
            csum = _psum16(_ones(m))
            pos = jnp.where(m, carries[e] + csum - 1, pos)
            carries[e] = carries[e] + _splat_last(csum)

        vals = vals_v[pl.ds(j * L, L)]
        keep = pos < CAP
        dst = v * CAP + pos
        cidx_v[pl.ds(j * L, L)] = jnp.where(keep, dst, v * CAP + (CAP - 1))
        scale_v[pl.ds(j * L, L)] = jnp.where(keep, vals, 0.0)
        dstv = jnp.where(keep, dst, TRASH)
        tokv = lax.shift_right_logical(
            jnp.full((L,), base + j * L, jnp.int32) + lanei, 1)
        s, q = divmod(j, NVEC // NSUB)
        tref = tok0_v if s % 2 == 0 else tok1_v
        dref = dst0_v if s % 2 == 0 else dst1_v
        tref[pl.ds(q * L, L)] = tokv
        dref[pl.ds(q * L, L)] = dstv
        if q == NVEC // NSUB - 1:
            rref = rows0_v if s % 2 == 0 else rows1_v
            pltpu.async_copy(x_hbm.at[tref], rref, gsem).wait()
            pltpu.async_copy(rref, buf_hbm.at[dref], ssem).wait()

    @pl.when(wid == NW - 1)
    def _():
        tot = zeros
        for e in range(E):
            tot = jnp.where(lanei == e, carries[e], tot)
        cnt_v[...] = tot
        pltpu.sync_copy(cnt_v, counts_hbm)

    pltpu.sync_copy(cidx_v, cidx_hbm.at[pl.ds(base, CHUNK)])
    pltpu.sync_copy(scale_v, scale_hbm.at[pl.ds(base, CHUNK)])


@functools.partial(
    pl.kernel,
    out_type=jax.ShapeDtypeStruct((P, H), jnp.float32),
    mesh=_MESH,
    scratch_types=(
        pltpu.VMEM((CHUNK,), jnp.int32),
        pltpu.VMEM((SUB,), jnp.int32),
        pltpu.VMEM((SUB,), jnp.int32),
        pltpu.VMEM((SUB, H), jnp.float32),
        pltpu.VMEM((SUB, H), jnp.float32),
        pltpu.SemaphoreType.DMA,
    ),
)
def _gather_sc(outbuf_hbm, cidx_hbm, gathered_hbm,
               cidx_v, sidx0_v, sidx1_v, rows0_v, rows1_v, gsem):
    wid = lax.axis_index("s") * NC + lax.axis_index("c")
    base = wid * CHUNK
    pltpu.sync_copy(cidx_hbm.at[pl.ds(base, CHUNK)], cidx_v)
    for s in range(NSUB):
        sref = sidx0_v if s % 2 == 0 else sidx1_v
        rref = rows0_v if s % 2 == 0 else rows1_v
        for q in range(SUB // L):
            sref[pl.ds(q * L, L)] = cidx_v[pl.ds(s * SUB + q * L, L)]
        pltpu.async_copy(outbuf_hbm.at[sref], rref, gsem).wait()
        pltpu.sync_copy(rref, gathered_hbm.at[pl.ds(base + s * SUB, SUB)])


def _dense_body(counts_ref, buf_ref, w1_ref, w3_ref, w2_ref, out_ref, acc_ref):
    e = pl.program_id(0)
    i = pl.program_id(1)
    cnt = counts_ref[e]
    nrb = (cnt + BR - 1) // BR

    @pl.when(i == 0)
    def _():
        acc_ref[...] = jnp.zeros_like(acc_ref)

    def body(rb, carry):
        rows = buf_ref[pl.ds(rb * BR, BR), :].astype(jnp.bfloat16)
        gate = jnp.dot(rows, w1_ref[0], preferred_element_type=jnp.float32)
        up = jnp.dot(rows, w3_ref[0], preferred_element_type=jnp.float32)
        act = (gate * jax.nn.sigmoid(gate) * up).astype(jnp.bfloat16)
        partial = jnp.dot(act, w2_ref[0], preferred_element_type=jnp.float32)
        acc_ref[pl.ds(rb * BR, BR), :] += partial
        return carry

    lax.fori_loop(0, nrb, body, 0)

    @pl.when(i == NI - 1)
    def _():
        out_ref[...] = acc_ref[...]


def _dense(buf, counts, w1b, w3b, w2b):
    return pl.pallas_call(
        _dense_body,
        grid=(E, NI),
        in_specs=[
            pl.BlockSpec(memory_space=pltpu.SMEM),
            pl.BlockSpec((CAP, H), lambda e, i: (e, 0)),
            pl.BlockSpec((1, H, BI), lambda e, i: (e, 0, i)),
            pl.BlockSpec((1, H, BI), lambda e, i: (e, 0, i)),
            pl.BlockSpec((1, BI, H), lambda e, i: (e, i, 0)),
        ],
        out_specs=pl.BlockSpec((CAP, H), lambda e, i: (e, 0)),
        out_shape=jax.ShapeDtypeStruct((E * CAP, H), jnp.float32),
        scratch_shapes=[pltpu.VMEM((CAP, H), jnp.float32)],
        compiler_params=pltpu.CompilerParams(
            dimension_semantics=("arbitrary", "arbitrary"),
        ),
    )(counts, buf, w1b, w3b, w2b)


BT = 128


def _combine_body(g_ref, s_ref, o_ref):
    gs = g_ref[...] * s_ref[...][:, 0:1]
    o_ref[...] = gs.reshape(BT, K, H).sum(axis=1)


def _combine(gathered, scale2d):
    return pl.pallas_call(
        _combine_body,
        grid=(T // BT,),
        in_specs=[
            pl.BlockSpec((BT * K, H), lambda t: (t, 0)),
            pl.BlockSpec((BT * K, 128), lambda t: (t, 0)),
        ],
        out_specs=pl.BlockSpec((BT, H), lambda t: (t, 0)),
        out_shape=jax.ShapeDtypeStruct((T, H), jnp.float32),
    )(gathered, scale2d)


def kernel(hidden_states, top_k_indices, top_k_values, w1, w2, w3):
    idx_flat = top_k_indices.reshape(-1)
    vals_flat = top_k_values.reshape(-1)
    buf, counts16, cidx, scale = _dispatch_sc(
        idx_flat, vals_flat, hidden_states)
    out_buf = _dense(buf, counts16, w1.astype(jnp.bfloat16),
                     w3.astype(jnp.bfloat16), w2.astype(jnp.bfloat16))
    gathered = _gather_sc(out_buf, cidx)
    scale2d = jnp.broadcast_to(scale[:, None], (P, 128))
    return _combine(gathered, scale2d)


# speedup vs baseline: 1.8978x; 1.5382x over previous
import functools

import jax
import jax.numpy as jnp
from jax import lax
from jax.experimental import pallas as pl
from jax.experimental.pallas import tpu as pltpu
from jax.experimental.pallas import tpu_sc as plsc

E = 8
H = 1024
I = 4096
K = 2
T = 2048
CAP = 1024
P = T * K
TRASH = E * CAP
BUF_ROWS = E * CAP + 8

BI = 1024
NI = I // BI
BR = 256

NC = 2
NS = 16
L = 16
NW = NC * NS
CHUNK = P // NW
NVEC = CHUNK // L
SUB = 32
NSUB = CHUNK // SUB

_MESH = plsc.VectorSubcoreMesh(core_axis_name="c", subcore_axis_name="s")


def _iota16():
    return lax.iota(jnp.int32, L)


def _ones(m):
    return jnp.where(m, 1, 0)


def _psum16(c):
    lane = _iota16()
    for k in (1, 2, 4, 8):
        sh = c[jnp.maximum(lane - k, 0)]
        c = c + jnp.where(lane >= k, sh, 0)
    return c


def _splat_last(c):
    return c[jnp.full((L,), L - 1, jnp.int32)]


@functools.partial(
    pl.kernel,
    out_type=(
        jax.ShapeDtypeStruct((BUF_ROWS, H), jnp.float32),
        jax.ShapeDtypeStruct((16,), jnp.int32),
        jax.ShapeDtypeStruct((P,), jnp.int32),
        jax.ShapeDtypeStruct((P,), jnp.float32),
    ),
    mesh=_MESH,
    scratch_types=(
        pltpu.VMEM((P,), jnp.int32),
        pltpu.VMEM((CHUNK,), jnp.float32),
        pltpu.VMEM((CHUNK,), jnp.int32),
        pltpu.VMEM((CHUNK,), jnp.float32),
        pltpu.VMEM((16,), jnp.int32),
        pltpu.VMEM((SUB,), jnp.int32),
        pltpu.VMEM((SUB,), jnp.int32),
        pltpu.VMEM((SUB,), jnp.int32),
        pltpu.VMEM((SUB,), jnp.int32),
        pltpu.VMEM((SUB, H), jnp.float32),
        pltpu.VMEM((SUB, H), jnp.float32),
        pltpu.SemaphoreType.DMA,
        pltpu.SemaphoreType.DMA,
    ),
)
def _dispatch_sc(idx_hbm, vals_hbm, x_hbm,
                 buf_hbm, counts_hbm, cidx_hbm, scale_hbm,
                 idx_v, vals_v, cidx_v, scale_v, cnt_v,
                 tok0_v, dst0_v, tok1_v, dst1_v, rows0_v, rows1_v,
                 gsem, ssem):
    wid = lax.axis_index("s") * NC + lax.axis_index("c")
    base = wid * CHUNK

    pltpu.sync_copy(idx_hbm, idx_v)
    pltpu.sync_copy(vals_hbm.at[pl.ds(base, CHUNK)], vals_v)

    zeros = jnp.zeros((L,), jnp.int32)
    lanei = _iota16()

    def prefix_body(i, cs):
        v = idx_v[pl.ds(i * L, L)]
        return tuple(cs[e] + _ones(v == e) for e in range(E))

    carry_vecs = lax.fori_loop(0, wid * NVEC, prefix_body,
                               tuple(zeros for _ in range(E)))
    carries = [_splat_last(_psum16(carry_vecs[e])) for e in range(E)]

    for j in range(NVEC):
        v = idx_v[pl.ds(base + j * L, L)]
        pos = zeros
        for e in range(E):
            m = v == e
            csum = _psum16(_ones(m))
            pos = jnp.where(m, carries[e] + csum - 1, pos)
            carries[e] = carries[e] + _splat_last(csum)

        vals = vals_v[pl.ds(j * L, L)]
        keep = pos < CAP
        dst = v * CAP + pos
        cidx_v[pl.ds(j * L, L)] = jnp.where(keep, dst, v * CAP + (CAP - 1))
        scale_v[pl.ds(j * L, L)] = jnp.where(keep, vals, 0.0)
        dstv = jnp.where(keep, dst, TRASH)
        tokv = lax.shift_right_logical(
            jnp.full((L,), base + j * L, jnp.int32) + lanei, 1)
        s, q = divmod(j, NVEC // NSUB)
        tref = tok0_v if s % 2 == 0 else tok1_v
        dref = dst0_v if s % 2 == 0 else dst1_v
        tref[pl.ds(q * L, L)] = tokv
        dref[pl.ds(q * L, L)] = dstv
        if q == NVEC // NSUB - 1:
            rref = rows0_v if s % 2 == 0 else rows1_v
            pltpu.async_copy(x_hbm.at[tref], rref, gsem).wait()
            pltpu.async_copy(rref, buf_hbm.at[dref], ssem).wait()

    @pl.when(wid == NW - 1)
    def _():
        tot = zeros
        for e in range(E):
            tot = jnp.where(lanei == e, carries[e], tot)
        cnt_v[...] = tot
        pltpu.sync_copy(cnt_v, counts_hbm)

    pltpu.sync_copy(cidx_v, cidx_hbm.at[pl.ds(base, CHUNK)])
    pltpu.sync_copy(scale_v, scale_hbm.at[pl.ds(base, CHUNK)])


@functools.partial(
    pl.kernel,
    out_type=jax.ShapeDtypeStruct((P, H), jnp.float32),
    mesh=_MESH,
    scratch_types=(
        pltpu.VMEM((CHUNK,), jnp.int32),
        pltpu.VMEM((SUB,), jnp.int32),
        pltpu.VMEM((SUB,), jnp.int32),
        pltpu.VMEM((SUB, H), jnp.float32),
        pltpu.VMEM((SUB, H), jnp.float32),
        pltpu.SemaphoreType.DMA,
    ),
)
def _gather_sc(outbuf_hbm, cidx_hbm, gathered_hbm,
               cidx_v, sidx0_v, sidx1_v, rows0_v, rows1_v, gsem):
    wid = lax.axis_index("s") * NC + lax.axis_index("c")
    base = wid * CHUNK
    pltpu.sync_copy(cidx_hbm.at[pl.ds(base, CHUNK)], cidx_v)
    for s in range(NSUB):
        sref = sidx0_v if s % 2 == 0 else sidx1_v
        rref = rows0_v if s % 2 == 0 else rows1_v
        for q in range(SUB // L):
            sref[pl.ds(q * L, L)] = cidx_v[pl.ds(s * SUB + q * L, L)]
        pltpu.async_copy(outbuf_hbm.at[sref], rref, gsem).wait()
        pltpu.sync_copy(rref, gathered_hbm.at[pl.ds(base + s * SUB, SUB)])


def _dense_body(counts_ref, buf_ref, w1_ref, w3_ref, w2_ref, out_ref, acc_ref):
    e = pl.program_id(0)
    i = pl.program_id(1)
    cnt = counts_ref[e]
    nrb = (cnt + BR - 1) // BR

    @pl.when(i == 0)
    def _():
        acc_ref[...] = jnp.zeros_like(acc_ref)

    def body(rb, carry):
        rows = buf_ref[pl.ds(rb * BR, BR), :]
        gate = jnp.dot(rows, w1_ref[0], preferred_element_type=jnp.float32)
        up = jnp.dot(rows, w3_ref[0], preferred_element_type=jnp.float32)
        act = gate * jax.nn.sigmoid(gate) * up
        partial = jnp.dot(act, w2_ref[0], preferred_element_type=jnp.float32)
        acc_ref[pl.ds(rb * BR, BR), :] += partial
        return carry

    lax.fori_loop(0, nrb, body, 0)

    @pl.when(i == NI - 1)
    def _():
        out_ref[...] = acc_ref[...]


def _dense(buf, counts, w1b, w3b, w2b):
    return pl.pallas_call(
        _dense_body,
        grid=(E, NI),
        in_specs=[
            pl.BlockSpec(memory_space=pltpu.SMEM),
            pl.BlockSpec((CAP, H), lambda e, i: (e, 0)),
            pl.BlockSpec((1, H, BI), lambda e, i: (e, 0, i)),
            pl.BlockSpec((1, H, BI), lambda e, i: (e, 0, i)),
            pl.BlockSpec((1, BI, H), lambda e, i: (e, i, 0)),
        ],
        out_specs=pl.BlockSpec((CAP, H), lambda e, i: (e, 0)),
        out_shape=jax.ShapeDtypeStruct((E * CAP, H), jnp.float32),
        scratch_shapes=[pltpu.VMEM((CAP, H), jnp.float32)],
        compiler_params=pltpu.CompilerParams(
            dimension_semantics=("arbitrary", "arbitrary"),
        ),
    )(counts, buf, w1b, w3b, w2b)


BT = 128


def _combine_body(g_ref, s_ref, o_ref):
    gs = g_ref[...] * s_ref[...][:, 0:1]
    o_ref[...] = gs.reshape(BT, K, H).sum(axis=1)


def _combine(gathered, scale2d):
    return pl.pallas_call(
        _combine_body,
        grid=(T // BT,),
        in_specs=[
            pl.BlockSpec((BT * K, H), lambda t: (t, 0)),
            pl.BlockSpec((BT * K, 128), lambda t: (t, 0)),
        ],
        out_specs=pl.BlockSpec((BT, H), lambda t: (t, 0)),
        out_shape=jax.ShapeDtypeStruct((T, H), jnp.float32),
    )(gathered, scale2d)


def kernel(hidden_states, top_k_indices, top_k_values, w1, w2, w3):
    idx_flat = top_k_indices.reshape(-1)
    vals_flat = top_k_values.reshape(-1)
    buf, counts16, cidx, scale = _dispatch_sc(
        idx_flat, vals_flat, hidden_states)
    out_buf = _dense(buf, counts16, w1, w3, w2)
    gathered = _gather_sc(out_buf, cidx)
    scale2d = jnp.broadcast_to(scale[:, None], (P, 128))
    return _combine(gathered, scale2d)


# speedup vs baseline: 2.0079x; 1.0580x over previous
import functools

import jax
import jax.numpy as jnp
from jax import lax
from jax.experimental import pallas as pl
from jax.experimental.pallas import tpu as pltpu
from jax.experimental.pallas import tpu_sc as plsc

E = 8
H = 1024
I = 4096
K = 2
T = 2048
CAP = 1024
P = T * K
TRASH = E * CAP
BUF_ROWS = E * CAP + 8

BI = 1024
NI = I // BI
BR = 256

NC = 2
NS = 16
L = 16
NW = NC * NS
CHUNK = P // NW
NVEC = CHUNK // L
SUB = 32
NSUB = CHUNK // SUB

_MESH = plsc.VectorSubcoreMesh(core_axis_name="c", subcore_axis_name="s")


def _iota16():
    return lax.iota(jnp.int32, L)


def _ones(m):
    return jnp.where(m, 1, 0)


def _psum16(c):
    lane = _iota16()
    for k in (1, 2, 4, 8):
        sh = c[jnp.maximum(lane - k, 0)]
        c = c + jnp.where(lane >= k, sh, 0)
    return c


def _splat_last(c):
    return c[jnp.full((L,), L - 1, jnp.int32)]


@functools.partial(
    pl.kernel,
    out_type=(
        jax.ShapeDtypeStruct((BUF_ROWS, H), jnp.float32),
        jax.ShapeDtypeStruct((16,), jnp.int32),
        jax.ShapeDtypeStruct((P,), jnp.int32),
        jax.ShapeDtypeStruct((P,), jnp.float32),
    ),
    mesh=_MESH,
    scratch_types=(
        pltpu.VMEM((P,), jnp.int32),
        pltpu.VMEM((CHUNK,), jnp.float32),
        pltpu.VMEM((CHUNK,), jnp.int32),
        pltpu.VMEM((CHUNK,), jnp.float32),
        pltpu.VMEM((16,), jnp.int32),
        pltpu.VMEM((SUB,), jnp.int32),
        pltpu.VMEM((SUB,), jnp.int32),
        pltpu.VMEM((SUB,), jnp.int32),
        pltpu.VMEM((SUB,), jnp.int32),
        pltpu.VMEM((SUB, H), jnp.float32),
        pltpu.VMEM((SUB, H), jnp.float32),
        pltpu.SemaphoreType.DMA,
        pltpu.SemaphoreType.DMA,
    ),
)
def _dispatch_sc(idx_hbm, vals_hbm, x_hbm,
                 buf_hbm, counts_hbm, cidx_hbm, scale_hbm,
                 idx_v, vals_v, cidx_v, scale_v, cnt_v,
                 tok0_v, dst0_v, tok1_v, dst1_v, rows0_v, rows1_v,
                 gsem, ssem):
    wid = lax.axis_index("s") * NC + lax.axis_index("c")
    base = wid * CHUNK

    pltpu.sync_copy(idx_hbm, idx_v)
    pltpu.sync_copy(vals_hbm.at[pl.ds(base, CHUNK)], vals_v)

    zeros = jnp.zeros((L,), jnp.int32)
    lanei = _iota16()

    def prefix_body(i, cs):
        v = idx_v[pl.ds(i * L, L)]
        return tuple(cs[e] + _ones(v == e) for e in range(E))

    carry_vecs = lax.fori_loop(0, wid * NVEC, prefix_body,
                               tuple(zeros for _ in range(E)))
    carries = [_splat_last(_psum16(carry_vecs[e])) for e in range(E)]

    for j in range(NVEC):
        v = idx_v[pl.ds(base + j * L, L)]
        pos = zeros
        for e in range(E):
            m = v == e
            csum = _psum16(_ones(m))
            pos = jnp.where(m, carries[e] + csum - 1, pos)
            carries[e] = carries[e] + _splat_last(csum)

        vals = vals_v[pl.ds(j * L, L)]
        keep = pos < CAP
        dst = v * CAP + pos
        cidx_v[pl.ds(j * L, L)] = jnp.where(keep, dst, v * CAP + (CAP - 1))
        scale_v[pl.ds(j * L, L)] = jnp.where(keep, vals, 0.0)
        dstv = jnp.where(keep, dst, TRASH)
        tokv = lax.shift_right_logical(
            jnp.full((L,), base + j * L, jnp.int32) + lanei, 1)
        s, q = divmod(j, NVEC // NSUB)
        tref = tok0_v if s % 2 == 0 else tok1_v
        dref = dst0_v if s % 2 == 0 else dst1_v
        tref[pl.ds(q * L, L)] = tokv
        dref[pl.ds(q * L, L)] = dstv
        if q == NVEC // NSUB - 1:
            rref = rows0_v if s % 2 == 0 else rows1_v
            pltpu.async_copy(x_hbm.at[tref], rref, gsem).wait()
            pltpu.async_copy(rref, buf_hbm.at[dref], ssem).wait()

    @pl.when(wid == NW - 1)
    def _():
        tot = zeros
        for e in range(E):
            tot = jnp.where(lanei == e, carries[e], tot)
        cnt_v[...] = tot
        pltpu.sync_copy(cnt_v, counts_hbm)

    pltpu.sync_copy(cidx_v, cidx_hbm.at[pl.ds(base, CHUNK)])
    pltpu.sync_copy(scale_v, scale_hbm.at[pl.ds(base, CHUNK)])


@functools.partial(
    pl.kernel,
    out_type=(
        jax.ShapeDtypeStruct((T, H), jnp.float32),
        jax.ShapeDtypeStruct((T, H), jnp.float32),
    ),
    mesh=_MESH,
    scratch_types=(
        pltpu.VMEM((CHUNK,), jnp.int32),
        pltpu.VMEM((SUB,), jnp.int32),
        pltpu.VMEM((SUB,), jnp.int32),
        pltpu.VMEM((SUB, H), jnp.float32),
        pltpu.VMEM((SUB, H), jnp.float32),
        pltpu.SemaphoreType.DMA,
    ),
)
def _gather_sc(outbuf_hbm, cidx_hbm, g0_hbm, g1_hbm,
               cidx_v, sidx0_v, sidx1_v, rows0_v, rows1_v, gsem):
    wid = lax.axis_index("s") * NC + lax.axis_index("c")
    base = wid * CHUNK
    tbase = wid * (CHUNK // K)
    lane = _iota16()
    pltpu.sync_copy(cidx_hbm.at[pl.ds(base, CHUNK)], cidx_v)
    for s in range(NSUB):
        sref = sidx0_v if s % 2 == 0 else sidx1_v
        rref = rows0_v if s % 2 == 0 else rows1_v
        v0 = cidx_v[pl.ds(s * SUB, L)]
        v1 = cidx_v[pl.ds(s * SUB + L, L)]
        half = lane < (L // 2)
        i2 = (lane + lane) & (L - 1)
        i2o = (lane + lane + 1) & (L - 1)
        sref[pl.ds(0, L)] = jnp.where(half, v0[i2], v1[i2])
        sref[pl.ds(L, L)] = jnp.where(half, v0[i2o], v1[i2o])
        pltpu.async_copy(outbuf_hbm.at[sref], rref, gsem).wait()
        nt = SUB // K
        pltpu.sync_copy(rref.at[pl.ds(0, nt)],
                        g0_hbm.at[pl.ds(tbase + s * nt, nt)])
        pltpu.sync_copy(rref.at[pl.ds(nt, nt)],
                        g1_hbm.at[pl.ds(tbase + s * nt, nt)])


def _dense_body(counts_ref, buf_ref, w1_ref, w3_ref, w2_ref, out_ref, acc_ref):
    e = pl.program_id(0)
    i = pl.program_id(1)
    cnt = counts_ref[e]
    nrb = (cnt + BR - 1) // BR

    @pl.when(i == 0)
    def _():
        acc_ref[...] = jnp.zeros_like(acc_ref)

    def body(rb, carry):
        rows = buf_ref[pl.ds(rb * BR, BR), :]
        gate = jnp.dot(rows, w1_ref[0], preferred_element_type=jnp.float32)
        up = jnp.dot(rows, w3_ref[0], preferred_element_type=jnp.float32)
        act = gate * jax.nn.sigmoid(gate) * up
        partial = jnp.dot(act, w2_ref[0], preferred_element_type=jnp.float32)
        acc_ref[pl.ds(rb * BR, BR), :] += partial
        return carry

    lax.fori_loop(0, nrb, body, 0)

    @pl.when(i == NI - 1)
    def _():
        out_ref[...] = acc_ref[...]


def _dense(buf, counts, w1b, w3b, w2b):
    return pl.pallas_call(
        _dense_body,
        grid=(E, NI),
        in_specs=[
            pl.BlockSpec(memory_space=pltpu.SMEM),
            pl.BlockSpec((CAP, H), lambda e, i: (e, 0)),
            pl.BlockSpec((1, H, BI), lambda e, i: (e, 0, i)),
            pl.BlockSpec((1, H, BI), lambda e, i: (e, 0, i)),
            pl.BlockSpec((1, BI, H), lambda e, i: (e, i, 0)),
        ],
        out_specs=pl.BlockSpec((CAP, H), lambda e, i: (e, 0)),
        out_shape=jax.ShapeDtypeStruct((E * CAP, H), jnp.float32),
        scratch_shapes=[pltpu.VMEM((CAP, H), jnp.float32)],
        compiler_params=pltpu.CompilerParams(
            dimension_semantics=("arbitrary", "arbitrary"),
        ),
    )(counts, buf, w1b, w3b, w2b)


BT = 256


def _combine_body(g0_ref, g1_ref, s0_ref, s1_ref, o_ref):
    o_ref[...] = (g0_ref[...] * s0_ref[...][:, 0:1]
                  + g1_ref[...] * s1_ref[...][:, 0:1])


def _combine(g0, g1, s0col, s1col):
    return pl.pallas_call(
        _combine_body,
        grid=(T // BT,),
        in_specs=[
            pl.BlockSpec((BT, H), lambda t: (t, 0)),
            pl.BlockSpec((BT, H), lambda t: (t, 0)),
            pl.BlockSpec((BT, 128), lambda t: (t, 0)),
            pl.BlockSpec((BT, 128), lambda t: (t, 0)),
        ],
        out_specs=pl.BlockSpec((BT, H), lambda t: (t, 0)),
        out_shape=jax.ShapeDtypeStruct((T, H), jnp.float32),
    )(g0, g1, s0col, s1col)


def kernel(hidden_states, top_k_indices, top_k_values, w1, w2, w3):
    idx_flat = top_k_indices.reshape(-1)
    vals_flat = top_k_values.reshape(-1)
    buf, counts16, cidx, scale = _dispatch_sc(
        idx_flat, vals_flat, hidden_states)
    out_buf = _dense(buf, counts16, w1, w3, w2)
    g0, g1 = _gather_sc(out_buf, cidx)
    s01 = scale.reshape(T, K)
    s0col = jnp.broadcast_to(s01[:, 0:1], (T, 128))
    s1col = jnp.broadcast_to(s01[:, 1:2], (T, 128))
    return _combine(g0, g1, s0col, s1col)


# speedup vs baseline: 2.0797x; 1.0358x over previous
import functools

import jax
import jax.numpy as jnp
from jax import lax
from jax.experimental import pallas as pl
from jax.experimental.pallas import tpu as pltpu
from jax.experimental.pallas import tpu_sc as plsc

E = 8
H = 1024
I = 4096
K = 2
T = 2048
CAP = 1024
P = T * K
TRASH = E * CAP
BUF_ROWS = E * CAP + 8

BI = 1024
NI = I // BI
BR = 256

NC = 2
NS = 16
L = 16
NW = NC * NS
CHUNK = P // NW
TPW = CHUNK // K
NVEC = CHUNK // L
SUB = 32
NSUB = CHUNK // SUB

_MESH = plsc.VectorSubcoreMesh(core_axis_name="c", subcore_axis_name="s")


def _iota16():
    return lax.iota(jnp.int32, L)


def _ones(m):
    return jnp.where(m, 1, 0)


def _psum16(c):
    lane = _iota16()
    for k in (1, 2, 4, 8):
        sh = c[jnp.maximum(lane - k, 0)]
        c = c + jnp.where(lane >= k, sh, 0)
    return c


def _splat_last(c):
    return c[jnp.full((L,), L - 1, jnp.int32)]


@functools.partial(
    pl.kernel,
    out_type=(
        jax.ShapeDtypeStruct((BUF_ROWS, H), jnp.float32),
        jax.ShapeDtypeStruct((16,), jnp.int32),
        jax.ShapeDtypeStruct((P,), jnp.int32),
        jax.ShapeDtypeStruct((P,), jnp.float32),
    ),
    mesh=_MESH,
    scratch_types=(
        pltpu.VMEM((P,), jnp.int32),
        pltpu.VMEM((CHUNK,), jnp.float32),
        pltpu.VMEM((CHUNK,), jnp.int32),
        pltpu.VMEM((CHUNK,), jnp.float32),
        pltpu.VMEM((16,), jnp.int32),
        pltpu.VMEM((TPW // 2,), jnp.int32),
        pltpu.VMEM((TPW // 2,), jnp.int32),
        pltpu.VMEM((TPW // 2,), jnp.int32),
        pltpu.VMEM((TPW // 2,), jnp.int32),
        pltpu.VMEM((TPW, H), jnp.float32),
        pltpu.SemaphoreType.DMA,
        pltpu.SemaphoreType.DMA,
    ),
)
def _dispatch_sc(idx_hbm, vals_hbm, x_hbm,
                 buf_hbm, counts_hbm, cidx_hbm, scale_hbm,
                 idx_v, vals_v, cidx_v, scale_v, cnt_v,
                 dste0_v, dsto0_v, dste1_v, dsto1_v, rows_v,
                 xsem, ssem):
    wid = lax.axis_index("s") * NC + lax.axis_index("c")
    base = wid * CHUNK
    tbase = wid * TPW

    xcopy = pltpu.async_copy(x_hbm.at[pl.ds(tbase, TPW)], rows_v, xsem)

    pltpu.sync_copy(idx_hbm, idx_v)
    pltpu.sync_copy(vals_hbm.at[pl.ds(base, CHUNK)], vals_v)

    zeros = jnp.zeros((L,), jnp.int32)
    lanei = _iota16()
    half = lanei < (L // 2)
    i2 = (lanei + lanei) & (L - 1)
    i2o = (lanei + lanei + 1) & (L - 1)

    def prefix_body(i, cs):
        v = idx_v[pl.ds(i * L, L)]
        return tuple(cs[e] + _ones(v == e) for e in range(E))

    carry_vecs = lax.fori_loop(0, wid * NVEC, prefix_body,
                               tuple(zeros for _ in range(E)))
    carries = [_splat_last(_psum16(carry_vecs[e])) for e in range(E)]

    dstvs = []
    for j in range(NVEC):
        v = idx_v[pl.ds(base + j * L, L)]
        pos = zeros
        for e in range(E):
            m = v == e
            csum = _psum16(_ones(m))
            pos = jnp.where(m, carries[e] + csum - 1, pos)
            carries[e] = carries[e] + _splat_last(csum)

        vals = vals_v[pl.ds(j * L, L)]
        keep = pos < CAP
        dst = v * CAP + pos
        cidx_v[pl.ds(j * L, L)] = jnp.where(keep, dst, v * CAP + (CAP - 1))
        scale_v[pl.ds(j * L, L)] = jnp.where(keep, vals, 0.0)
        dstvs.append(jnp.where(keep, dst, TRASH))

    for b, (eref, oref) in ((0, (dste0_v, dsto0_v)), (1, (dste1_v, dsto1_v))):
        for h in range(2):
            d0 = dstvs[4 * b + 2 * h]
            d1 = dstvs[4 * b + 2 * h + 1]
            eref[pl.ds(h * L, L)] = jnp.where(half, d0[i2], d1[i2])
            oref[pl.ds(h * L, L)] = jnp.where(half, d0[i2o], d1[i2o])

    pltpu.sync_copy(cidx_v, cidx_hbm.at[pl.ds(base, CHUNK)])
    pltpu.sync_copy(scale_v, scale_hbm.at[pl.ds(base, CHUNK)])

    @pl.when(wid == NW - 1)
    def _():
        tot = zeros
        for e in range(E):
            tot = jnp.where(lanei == e, carries[e], tot)
        cnt_v[...] = tot
        pltpu.sync_copy(cnt_v, counts_hbm)

    xcopy.wait()
    hb = TPW // 2
    scs = [
        pltpu.async_copy(rows_v.at[pl.ds(0, hb)], buf_hbm.at[dste0_v], ssem),
        pltpu.async_copy(rows_v.at[pl.ds(0, hb)], buf_hbm.at[dsto0_v], ssem),
        pltpu.async_copy(rows_v.at[pl.ds(hb, hb)], buf_hbm.at[dste1_v], ssem),
        pltpu.async_copy(rows_v.at[pl.ds(hb, hb)], buf_hbm.at[dsto1_v], ssem),
    ]
    for sc in scs:
        sc.wait()


@functools.partial(
    pl.kernel,
    out_type=(
        jax.ShapeDtypeStruct((T, H), jnp.float32),
        jax.ShapeDtypeStruct((T, H), jnp.float32),
    ),
    mesh=_MESH,
    scratch_types=(
        pltpu.VMEM((CHUNK,), jnp.int32),
        pltpu.VMEM((SUB,), jnp.int32),
        pltpu.VMEM((SUB,), jnp.int32),
        pltpu.VMEM((SUB, H), jnp.float32),
        pltpu.VMEM((SUB, H), jnp.float32),
        pltpu.SemaphoreType.DMA,
    ),
)
def _gather_sc(outbuf_hbm, cidx_hbm, g0_hbm, g1_hbm,
               cidx_v, sidx0_v, sidx1_v, rows0_v, rows1_v, gsem):
    wid = lax.axis_index("s") * NC + lax.axis_index("c")
    base = wid * CHUNK
    tbase = wid * (CHUNK // K)
    lane = _iota16()
    pltpu.sync_copy(cidx_hbm.at[pl.ds(base, CHUNK)], cidx_v)
    for s in range(NSUB):
        sref = sidx0_v if s % 2 == 0 else sidx1_v
        rref = rows0_v if s % 2 == 0 else rows1_v
        v0 = cidx_v[pl.ds(s * SUB, L)]
        v1 = cidx_v[pl.ds(s * SUB + L, L)]
        half = lane < (L // 2)
        i2 = (lane + lane) & (L - 1)
        i2o = (lane + lane + 1) & (L - 1)
        sref[pl.ds(0, L)] = jnp.where(half, v0[i2], v1[i2])
        sref[pl.ds(L, L)] = jnp.where(half, v0[i2o], v1[i2o])
        pltpu.async_copy(outbuf_hbm.at[sref], rref, gsem).wait()
        nt = SUB // K
        pltpu.sync_copy(rref.at[pl.ds(0, nt)],
                        g0_hbm.at[pl.ds(tbase + s * nt, nt)])
        pltpu.sync_copy(rref.at[pl.ds(nt, nt)],
                        g1_hbm.at[pl.ds(tbase + s * nt, nt)])


def _dense_body(counts_ref, buf_ref, w1_ref, w3_ref, w2_ref, out_ref, acc_ref):
    e = pl.program_id(0)
    i = pl.program_id(1)
    cnt = counts_ref[e]
    nrb = (cnt + BR - 1) // BR

    @pl.when(i == 0)
    def _():
        acc_ref[...] = jnp.zeros_like(acc_ref)

    def body(rb, carry):
        rows = buf_ref[pl.ds(rb * BR, BR), :]
        gate = jnp.dot(rows, w1_ref[0], preferred_element_type=jnp.float32)
        up = jnp.dot(rows, w3_ref[0], preferred_element_type=jnp.float32)
        act = gate * jax.nn.sigmoid(gate) * up
        partial = jnp.dot(act, w2_ref[0], preferred_element_type=jnp.float32)
        acc_ref[pl.ds(rb * BR, BR), :] += partial
        return carry

    lax.fori_loop(0, nrb, body, 0)

    @pl.when(i == NI - 1)
    def _():
        out_ref[...] = acc_ref[...]


def _dense(buf, counts, w1b, w3b, w2b):
    return pl.pallas_call(
        _dense_body,
        grid=(E, NI),
        in_specs=[
            pl.BlockSpec(memory_space=pltpu.SMEM),
            pl.BlockSpec((CAP, H), lambda e, i: (e, 0)),
            pl.BlockSpec((1, H, BI), lambda e, i: (e, 0, i)),
            pl.BlockSpec((1, H, BI), lambda e, i: (e, 0, i)),
            pl.BlockSpec((1, BI, H), lambda e, i: (e, i, 0)),
        ],
        out_specs=pl.BlockSpec((CAP, H), lambda e, i: (e, 0)),
        out_shape=jax.ShapeDtypeStruct((E * CAP, H), jnp.float32),
        scratch_shapes=[pltpu.VMEM((CAP, H), jnp.float32)],
        compiler_params=pltpu.CompilerParams(
            dimension_semantics=("arbitrary", "arbitrary"),
        ),
    )(counts, buf, w1b, w3b, w2b)


BT = 256


def _combine_body(g0_ref, g1_ref, s0_ref, s1_ref, o_ref):
    o_ref[...] = (g0_ref[...] * s0_ref[...][:, 0:1]
                  + g1_ref[...] * s1_ref[...][:, 0:1])


def _combine(g0, g1, s0col, s1col):
    return pl.pallas_call(
        _combine_body,
        grid=(T // BT,),
        in_specs=[
            pl.BlockSpec((BT, H), lambda t: (t, 0)),
            pl.BlockSpec((BT, H), lambda t: (t, 0)),
            pl.BlockSpec((BT, 128), lambda t: (t, 0)),
            pl.BlockSpec((BT, 128), lambda t: (t, 0)),
        ],
        out_specs=pl.BlockSpec((BT, H), lambda t: (t, 0)),
        out_shape=jax.ShapeDtypeStruct((T, H), jnp.float32),
    )(g0, g1, s0col, s1col)


def kernel(hidden_states, top_k_indices, top_k_values, w1, w2, w3):
    idx_flat = top_k_indices.reshape(-1)
    vals_flat = top_k_values.reshape(-1)
    buf, counts16, cidx, scale = _dispatch_sc(
        idx_flat, vals_flat, hidden_states)
    out_buf = _dense(buf, counts16, w1, w3, w2)
    g0, g1 = _gather_sc(out_buf, cidx)
    s01 = scale.reshape(T, K)
    s0col = jnp.broadcast_to(s01[:, 0:1], (T, 128))
    s1col = jnp.broadcast_to(s01[:, 1:2], (T, 128))
    return _combine(g0, g1, s0col, s1col)


# speedup vs baseline: 2.1044x; 1.0119x over previous
import functools

import jax
import jax.numpy as jnp
from jax import lax
from jax.experimental import pallas as pl
from jax.experimental.pallas import tpu as pltpu
from jax.experimental.pallas import tpu_sc as plsc

E = 8
H = 1024
I = 4096
K = 2
T = 2048
CAP = 1024
P = T * K
TRASH = E * CAP
BUF_ROWS = E * CAP + 8

BI = 1024
NI = I // BI
BR = 256

NC = 2
NS = 16
L = 16
NW = NC * NS
CHUNK = P // NW
TPW = CHUNK // K
NVEC = CHUNK // L
SUB = 32
NSUB = CHUNK // SUB

_MESH = plsc.VectorSubcoreMesh(core_axis_name="c", subcore_axis_name="s")


def _iota16():
    return lax.iota(jnp.int32, L)


def _ones(m):
    return jnp.where(m, 1, 0)


def _psum16(c):
    lane = _iota16()
    for k in (1, 2, 4, 8):
        sh = c[jnp.maximum(lane - k, 0)]
        c = c + jnp.where(lane >= k, sh, 0)
    return c


def _splat_last(c):
    return c[jnp.full((L,), L - 1, jnp.int32)]


@functools.partial(
    pl.kernel,
    out_type=(
        jax.ShapeDtypeStruct((BUF_ROWS, H), jnp.float32),
        jax.ShapeDtypeStruct((16,), jnp.int32),
        jax.ShapeDtypeStruct((P,), jnp.int32),
        jax.ShapeDtypeStruct((P,), jnp.float32),
    ),
    mesh=_MESH,
    scratch_types=(
        pltpu.VMEM((P,), jnp.int32),
        pltpu.VMEM((CHUNK,), jnp.float32),
        pltpu.VMEM((CHUNK,), jnp.int32),
        pltpu.VMEM((CHUNK,), jnp.float32),
        pltpu.VMEM((16,), jnp.int32),
        pltpu.VMEM((TPW // 2,), jnp.int32),
        pltpu.VMEM((TPW // 2,), jnp.int32),
        pltpu.VMEM((TPW // 2,), jnp.int32),
        pltpu.VMEM((TPW // 2,), jnp.int32),
        pltpu.VMEM((TPW, H), jnp.float32),
        pltpu.SemaphoreType.DMA,
        pltpu.SemaphoreType.DMA,
    ),
)
def _dispatch_sc(idx_hbm, vals_hbm, x_hbm,
                 buf_hbm, counts_hbm, cidx_hbm, scale_hbm,
                 idx_v, vals_v, cidx_v, scale_v, cnt_v,
                 dste0_v, dsto0_v, dste1_v, dsto1_v, rows_v,
                 xsem, ssem):
    wid = lax.axis_index("s") * NC + lax.axis_index("c")
    base = wid * CHUNK
    tbase = wid * TPW

    xcopy = pltpu.async_copy(x_hbm.at[pl.ds(tbase, TPW)], rows_v, xsem)

    pltpu.sync_copy(idx_hbm, idx_v)
    pltpu.sync_copy(vals_hbm.at[pl.ds(base, CHUNK)], vals_v)

    zeros = jnp.zeros((L,), jnp.int32)
    lanei = _iota16()
    half = lanei < (L // 2)
    i2 = (lanei + lanei) & (L - 1)
    i2o = (lanei + lanei + 1) & (L - 1)

    def prefix_body(i, cs):
        v = idx_v[pl.ds(i * L, L)]
        return tuple(cs[e] + _ones(v == e) for e in range(E))

    carry_vecs = lax.fori_loop(0, wid * NVEC, prefix_body,
                               tuple(zeros for _ in range(E)))
    carries = [_splat_last(_psum16(carry_vecs[e])) for e in range(E)]

    dstvs = []
    for j in range(NVEC):
        v = idx_v[pl.ds(base + j * L, L)]
        pos = zeros
        for e in range(E):
            m = v == e
            csum = _psum16(_ones(m))
            pos = jnp.where(m, carries[e] + csum - 1, pos)
            carries[e] = carries[e] + _splat_last(csum)

        vals = vals_v[pl.ds(j * L, L)]
        keep = pos < CAP
        dst = v * CAP + pos
        cidx_v[pl.ds(j * L, L)] = jnp.where(keep, dst, v * CAP + (CAP - 1))
        scale_v[pl.ds(j * L, L)] = jnp.where(keep, vals, 0.0)
        dstvs.append(jnp.where(keep, dst, TRASH))

    for b, (eref, oref) in ((0, (dste0_v, dsto0_v)), (1, (dste1_v, dsto1_v))):
        for h in range(2):
            d0 = dstvs[4 * b + 2 * h]
            d1 = dstvs[4 * b + 2 * h + 1]
            eref[pl.ds(h * L, L)] = jnp.where(half, d0[i2], d1[i2])
            oref[pl.ds(h * L, L)] = jnp.where(half, d0[i2o], d1[i2o])

    pltpu.sync_copy(cidx_v, cidx_hbm.at[pl.ds(base, CHUNK)])
    pltpu.sync_copy(scale_v, scale_hbm.at[pl.ds(base, CHUNK)])

    @pl.when(wid == NW - 1)
    def _():
        tot = zeros
        for e in range(E):
            tot = jnp.where(lanei == e, carries[e], tot)
        cnt_v[...] = tot
        pltpu.sync_copy(cnt_v, counts_hbm)

    xcopy.wait()
    hb = TPW // 2
    scs = [
        pltpu.async_copy(rows_v.at[pl.ds(0, hb)], buf_hbm.at[dste0_v], ssem),
        pltpu.async_copy(rows_v.at[pl.ds(0, hb)], buf_hbm.at[dsto0_v], ssem),
        pltpu.async_copy(rows_v.at[pl.ds(hb, hb)], buf_hbm.at[dste1_v], ssem),
        pltpu.async_copy(rows_v.at[pl.ds(hb, hb)], buf_hbm.at[dsto1_v], ssem),
    ]
    for sc in scs:
        sc.wait()


@functools.partial(
    pl.kernel,
    out_type=(
        jax.ShapeDtypeStruct((T, H), jnp.float32),
        jax.ShapeDtypeStruct((T, H), jnp.float32),
    ),
    mesh=_MESH,
    scratch_types=(
        pltpu.VMEM((CHUNK,), jnp.int32),
        pltpu.VMEM((SUB,), jnp.int32),
        pltpu.VMEM((SUB,), jnp.int32),
        pltpu.VMEM((SUB, H), jnp.float32),
        pltpu.VMEM((SUB, H), jnp.float32),
        pltpu.SemaphoreType.DMA,
        pltpu.SemaphoreType.DMA,
    ),
)
def _gather_sc(outbuf_hbm, cidx_hbm, g0_hbm, g1_hbm,
               cidx_v, sidx0_v, sidx1_v, rows0_v, rows1_v, gsem, osem):
    wid = lax.axis_index("s") * NC + lax.axis_index("c")
    base = wid * CHUNK
    tbase = wid * (CHUNK // K)
    lane = _iota16()
    half = lane < (L // 2)
    i2 = (lane + lane) & (L - 1)
    i2o = (lane + lane + 1) & (L - 1)
    nt = SUB // K
    pltpu.sync_copy(cidx_hbm.at[pl.ds(base, CHUNK)], cidx_v)

    def start_gather(s):
        sref = sidx0_v if s % 2 == 0 else sidx1_v
        rref = rows0_v if s % 2 == 0 else rows1_v
        v0 = cidx_v[pl.ds(s * SUB, L)]
        v1 = cidx_v[pl.ds(s * SUB + L, L)]
        sref[pl.ds(0, L)] = jnp.where(half, v0[i2], v1[i2])
        sref[pl.ds(L, L)] = jnp.where(half, v0[i2o], v1[i2o])
        return pltpu.async_copy(outbuf_hbm.at[sref], rref, gsem)

    def start_out(s):
        rref = rows0_v if s % 2 == 0 else rows1_v
        c0 = pltpu.async_copy(rref.at[pl.ds(0, nt)],
                              g0_hbm.at[pl.ds(tbase + s * nt, nt)], osem)
        c1 = pltpu.async_copy(rref.at[pl.ds(nt, nt)],
                              g1_hbm.at[pl.ds(tbase + s * nt, nt)], osem)
        return c0, c1

    g = [None] * NSUB
    o = [None] * NSUB
    g[0] = start_gather(0)
    g[1] = start_gather(1)
    for s in range(NSUB):
        g[s].wait()
        o[s] = start_out(s)
        if s + 2 < NSUB:
            for c in o[s]:
                c.wait()
            g[s + 2] = start_gather(s + 2)
    for s in range(NSUB - 2, NSUB):
        for c in o[s]:
            c.wait()


def _dense_body(counts_ref, buf_ref, w1_ref, w3_ref, w2_ref, out_ref, acc_ref):
    e = pl.program_id(0)
    i = pl.program_id(1)
    cnt = counts_ref[e]
    nrb = (cnt + BR - 1) // BR

    def make_body(first):
        def body(rb, carry):
            rows = buf_ref[pl.ds(rb * BR, BR), :]
            gate = jnp.dot(rows, w1_ref[0], preferred_element_type=jnp.float32)
            up = jnp.dot(rows, w3_ref[0], preferred_element_type=jnp.float32)
            act = gate * jax.nn.sigmoid(gate) * up
            part = jnp.dot(act, w2_ref[0], preferred_element_type=jnp.float32)
            if first:
                acc_ref[pl.ds(rb * BR, BR), :] = part
            else:
                acc_ref[pl.ds(rb * BR, BR), :] += part
            return carry
        return body

    @pl.when(i == 0)
    def _():
        lax.fori_loop(0, nrb, make_body(True), 0)

    @pl.when(i != 0)
    def _():
        lax.fori_loop(0, nrb, make_body(False), 0)

    @pl.when(i == NI - 1)
    def _():
        out_ref[...] = acc_ref[...]


def _dense(buf, counts, w1b, w3b, w2b):
    return pl.pallas_call(
        _dense_body,
        grid=(E, NI),
        in_specs=[
            pl.BlockSpec(memory_space=pltpu.SMEM),
            pl.BlockSpec((CAP, H), lambda e, i: (e, 0)),
            pl.BlockSpec((1, H, BI), lambda e, i: (e, 0, i)),
            pl.BlockSpec((1, H, BI), lambda e, i: (e, 0, i)),
            pl.BlockSpec((1, BI, H), lambda e, i: (e, i, 0)),
        ],
        out_specs=pl.BlockSpec((CAP, H), lambda e, i: (e, 0)),
        out_shape=jax.ShapeDtypeStruct((E * CAP, H), jnp.float32),
        scratch_shapes=[pltpu.VMEM((CAP, H), jnp.float32)],
        compiler_params=pltpu.CompilerParams(
            dimension_semantics=("arbitrary", "arbitrary"),
        ),
    )(counts, buf, w1b, w3b, w2b)


BT = 256


def _combine_body(g0_ref, g1_ref, s0_ref, s1_ref, o_ref):
    o_ref[...] = (g0_ref[...] * s0_ref[...][:, 0:1]
                  + g1_ref[...] * s1_ref[...][:, 0:1])


def _combine(g0, g1, s0col, s1col):
    return pl.pallas_call(
        _combine_body,
        grid=(T // BT,),
        in_specs=[
            pl.BlockSpec((BT, H), lambda t: (t, 0)),
            pl.BlockSpec((BT, H), lambda t: (t, 0)),
            pl.BlockSpec((BT, 128), lambda t: (t, 0)),
            pl.BlockSpec((BT, 128), lambda t: (t, 0)),
        ],
        out_specs=pl.BlockSpec((BT, H), lambda t: (t, 0)),
        out_shape=jax.ShapeDtypeStruct((T, H), jnp.float32),
    )(g0, g1, s0col, s1col)


def kernel(hidden_states, top_k_indices, top_k_values, w1, w2, w3):
    idx_flat = top_k_indices.reshape(-1)
    vals_flat = top_k_values.reshape(-1)
    buf, counts16, cidx, scale = _dispatch_sc(
        idx_flat, vals_flat, hidden_states)
    out_buf = _dense(buf, counts16, w1, w3, w2)
    g0, g1 = _gather_sc(out_buf, cidx)
    s01 = scale.reshape(T, K)
    s0col = jnp.broadcast_to(s01[:, 0:1], (T, 128))
    s1col = jnp.broadcast_to(s01[:, 1:2], (T, 128))
    return _combine(g0, g1, s0col, s1col)


# speedup vs baseline: 2.1066x; 1.0010x over previous
import functools

import jax
import jax.numpy as jnp
from jax import lax
from jax.experimental import pallas as pl
from jax.experimental.pallas import tpu as pltpu
from jax.experimental.pallas import tpu_sc as plsc

E = 8
H = 1024
I = 4096
K = 2
T = 2048
CAP = 1024
P = T * K
TRASH = E * CAP
BUF_ROWS = E * CAP + 8

BI = 1024
NI = I // BI
BR = 128

NC = 2
NS = 16
L = 16
NW = NC * NS
CHUNK = P // NW
TPW = CHUNK // K
NVEC = CHUNK // L
SUB = 32
NSUB = CHUNK // SUB

_MESH = plsc.VectorSubcoreMesh(core_axis_name="c", subcore_axis_name="s")


def _iota16():
    return lax.iota(jnp.int32, L)


def _ones(m):
    return jnp.where(m, 1, 0)


def _psum16(c):
    lane = _iota16()
    for k in (1, 2, 4, 8):
        sh = c[jnp.maximum(lane - k, 0)]
        c = c + jnp.where(lane >= k, sh, 0)
    return c


def _splat_last(c):
    return c[jnp.full((L,), L - 1, jnp.int32)]


@functools.partial(
    pl.kernel,
    out_type=(
        jax.ShapeDtypeStruct((BUF_ROWS, H), jnp.float32),
        jax.ShapeDtypeStruct((16,), jnp.int32),
        jax.ShapeDtypeStruct((P,), jnp.int32),
        jax.ShapeDtypeStruct((P,), jnp.float32),
    ),
    mesh=_MESH,
    scratch_types=(
        pltpu.VMEM((P,), jnp.int32),
        pltpu.VMEM((CHUNK,), jnp.float32),
        pltpu.VMEM((CHUNK,), jnp.int32),
        pltpu.VMEM((CHUNK,), jnp.float32),
        pltpu.VMEM((16,), jnp.int32),
        pltpu.VMEM((TPW // 2,), jnp.int32),
        pltpu.VMEM((TPW // 2,), jnp.int32),
        pltpu.VMEM((TPW // 2,), jnp.int32),
        pltpu.VMEM((TPW // 2,), jnp.int32),
        pltpu.VMEM((TPW, H), jnp.float32),
        pltpu.SemaphoreType.DMA,
        pltpu.SemaphoreType.DMA,
    ),
)
def _dispatch_sc(idx_hbm, vals_hbm, x_hbm,
                 buf_hbm, counts_hbm, cidx_hbm, scale_hbm,
                 idx_v, vals_v, cidx_v, scale_v, cnt_v,
                 dste0_v, dsto0_v, dste1_v, dsto1_v, rows_v,
                 xsem, ssem):
    wid = lax.axis_index("s") * NC + lax.axis_index("c")
    base = wid * CHUNK
    tbase = wid * TPW

    xcopy = pltpu.async_copy(x_hbm.at[pl.ds(tbase, TPW)], rows_v, xsem)

    pltpu.sync_copy(idx_hbm, idx_v)
    pltpu.sync_copy(vals_hbm.at[pl.ds(base, CHUNK)], vals_v)

    zeros = jnp.zeros((L,), jnp.int32)
    lanei = _iota16()
    half = lanei < (L // 2)
    i2 = (lanei + lanei) & (L - 1)
    i2o = (lanei + lanei + 1) & (L - 1)

    def prefix_body(i, cs):
        v = idx_v[pl.ds(i * L, L)]
        return tuple(cs[e] + _ones(v == e) for e in range(E))

    carry_vecs = lax.fori_loop(0, wid * NVEC, prefix_body,
                               tuple(zeros for _ in range(E)))
    carries = [_splat_last(_psum16(carry_vecs[e])) for e in range(E)]

    dstvs = []
    for j in range(NVEC):
        v = idx_v[pl.ds(base + j * L, L)]
        pos = zeros
        for e in range(E):
            m = v == e
            csum = _psum16(_ones(m))
            pos = jnp.where(m, carries[e] + csum - 1, pos)
            carries[e] = carries[e] + _splat_last(csum)

        vals = vals_v[pl.ds(j * L, L)]
        keep = pos < CAP
        dst = v * CAP + pos
        cidx_v[pl.ds(j * L, L)] = jnp.where(keep, dst, v * CAP + (CAP - 1))
        scale_v[pl.ds(j * L, L)] = jnp.where(keep, vals, 0.0)
        dstvs.append(jnp.where(keep, dst, TRASH))

    for b, (eref, oref) in ((0, (dste0_v, dsto0_v)), (1, (dste1_v, dsto1_v))):
        for h in range(2):
            d0 = dstvs[4 * b + 2 * h]
            d1 = dstvs[4 * b + 2 * h + 1]
            eref[pl.ds(h * L, L)] = jnp.where(half, d0[i2], d1[i2])
            oref[pl.ds(h * L, L)] = jnp.where(half, d0[i2o], d1[i2o])

    pltpu.sync_copy(cidx_v, cidx_hbm.at[pl.ds(base, CHUNK)])
    pltpu.sync_copy(scale_v, scale_hbm.at[pl.ds(base, CHUNK)])

    @pl.when(wid == NW - 1)
    def _():
        tot = zeros
        for e in range(E):
            tot = jnp.where(lanei == e, carries[e], tot)
        cnt_v[...] = tot
        pltpu.sync_copy(cnt_v, counts_hbm)

    xcopy.wait()
    hb = TPW // 2
    scs = [
        pltpu.async_copy(rows_v.at[pl.ds(0, hb)], buf_hbm.at[dste0_v], ssem),
        pltpu.async_copy(rows_v.at[pl.ds(0, hb)], buf_hbm.at[dsto0_v], ssem),
        pltpu.async_copy(rows_v.at[pl.ds(hb, hb)], buf_hbm.at[dste1_v], ssem),
        pltpu.async_copy(rows_v.at[pl.ds(hb, hb)], buf_hbm.at[dsto1_v], ssem),
    ]
    for sc in scs:
        sc.wait()


@functools.partial(
    pl.kernel,
    out_type=(
        jax.ShapeDtypeStruct((T, H), jnp.float32),
        jax.ShapeDtypeStruct((T, H), jnp.float32),
    ),
    mesh=_MESH,
    scratch_types=(
        pltpu.VMEM((CHUNK,), jnp.int32),
        pltpu.VMEM((SUB,), jnp.int32),
        pltpu.VMEM((SUB,), jnp.int32),
        pltpu.VMEM((SUB, H), jnp.float32),
        pltpu.VMEM((SUB, H), jnp.float32),
        pltpu.SemaphoreType.DMA,
        pltpu.SemaphoreType.DMA,
    ),
)
def _gather_sc(outbuf_hbm, cidx_hbm, g0_hbm, g1_hbm,
               cidx_v, sidx0_v, sidx1_v, rows0_v, rows1_v, gsem, osem):
    wid = lax.axis_index("s") * NC + lax.axis_index("c")
    base = wid * CHUNK
    tbase = wid * (CHUNK // K)
    lane = _iota16()
    half = lane < (L // 2)
    i2 = (lane + lane) & (L - 1)
    i2o = (lane + lane + 1) & (L - 1)
    nt = SUB // K
    pltpu.sync_copy(cidx_hbm.at[pl.ds(base, CHUNK)], cidx_v)

    def start_gather(s):
        sref = sidx0_v if s % 2 == 0 else sidx1_v
        rref = rows0_v if s % 2 == 0 else rows1_v
        v0 = cidx_v[pl.ds(s * SUB, L)]
        v1 = cidx_v[pl.ds(s * SUB + L, L)]
        sref[pl.ds(0, L)] = jnp.where(half, v0[i2], v1[i2])
        sref[pl.ds(L, L)] = jnp.where(half, v0[i2o], v1[i2o])
        return pltpu.async_copy(outbuf_hbm.at[sref], rref, gsem)

    def start_out(s):
        rref = rows0_v if s % 2 == 0 else rows1_v
        c0 = pltpu.async_copy(rref.at[pl.ds(0, nt)],
                              g0_hbm.at[pl.ds(tbase + s * nt, nt)], osem)
        c1 = pltpu.async_copy(rref.at[pl.ds(nt, nt)],
                              g1_hbm.at[pl.ds(tbase + s * nt, nt)], osem)
        return c0, c1

    g = [None] * NSUB
    o = [None] * NSUB
    g[0] = start_gather(0)
    g[1] = start_gather(1)
    for s in range(NSUB):
        g[s].wait()
        o[s] = start_out(s)
        if s + 2 < NSUB:
            for c in o[s]:
                c.wait()
            g[s + 2] = start_gather(s + 2)
    for s in range(NSUB - 2, NSUB):
        for c in o[s]:
            c.wait()


def _dense_body(counts_ref, buf_ref, w1_ref, w3_ref, w2_ref, out_ref, acc_ref):
    e = pl.program_id(0)
    i = pl.program_id(1)
    cnt = counts_ref[e]
    nrb = (cnt + BR - 1) // BR

    def make_body(first):
        def body(rb, carry):
            rows = buf_ref[pl.ds(rb * BR, BR), :]
            gate = jnp.dot(rows, w1_ref[0], preferred_element_type=jnp.float32)
            up = jnp.dot(rows, w3_ref[0], preferred_element_type=jnp.float32)
            act = gate * jax.nn.sigmoid(gate) * up
            part = jnp.dot(act, w2_ref[0], preferred_element_type=jnp.float32)
            if first:
                acc_ref[pl.ds(rb * BR, BR), :] = part
            else:
                acc_ref[pl.ds(rb * BR, BR), :] += part
            return carry
        return body

    @pl.when(i == 0)
    def _():
        lax.fori_loop(0, nrb, make_body(True), 0)

    @pl.when(i != 0)
    def _():
        lax.fori_loop(0, nrb, make_body(False), 0)

    @pl.when(i == NI - 1)
    def _():
        out_ref[...] = acc_ref[...]


def _dense(buf, counts, w1b, w3b, w2b):
    return pl.pallas_call(
        _dense_body,
        grid=(E, NI),
        in_specs=[
            pl.BlockSpec(memory_space=pltpu.SMEM),
            pl.BlockSpec((CAP, H), lambda e, i: (e, 0)),
            pl.BlockSpec((1, H, BI), lambda e, i: (e, 0, i)),
            pl.BlockSpec((1, H, BI), lambda e, i: (e, 0, i)),
            pl.BlockSpec((1, BI, H), lambda e, i: (e, i, 0)),
        ],
        out_specs=pl.BlockSpec((CAP, H), lambda e, i: (e, 0)),
        out_shape=jax.ShapeDtypeStruct((E * CAP, H), jnp.float32),
        scratch_shapes=[pltpu.VMEM((CAP, H), jnp.float32)],
        compiler_params=pltpu.CompilerParams(
            dimension_semantics=("arbitrary", "arbitrary"),
        ),
    )(counts, buf, w1b, w3b, w2b)


BT = 256


def _combine_body(g0_ref, g1_ref, s0_ref, s1_ref, o_ref):
    o_ref[...] = (g0_ref[...] * s0_ref[...][:, 0:1]
                  + g1_ref[...] * s1_ref[...][:, 0:1])


def _combine(g0, g1, s0col, s1col):
    return pl.pallas_call(
        _combine_body,
        grid=(T // BT,),
        in_specs=[
            pl.BlockSpec((BT, H), lambda t: (t, 0)),
            pl.BlockSpec((BT, H), lambda t: (t, 0)),
            pl.BlockSpec((BT, 128), lambda t: (t, 0)),
            pl.BlockSpec((BT, 128), lambda t: (t, 0)),
        ],
        out_specs=pl.BlockSpec((BT, H), lambda t: (t, 0)),
        out_shape=jax.ShapeDtypeStruct((T, H), jnp.float32),
    )(g0, g1, s0col, s1col)


def kernel(hidden_states, top_k_indices, top_k_values, w1, w2, w3):
    idx_flat = top_k_indices.reshape(-1)
    vals_flat = top_k_values.reshape(-1)
    buf, counts16, cidx, scale = _dispatch_sc(
        idx_flat, vals_flat, hidden_states)
    out_buf = _dense(buf, counts16, w1, w3, w2)
    g0, g1 = _gather_sc(out_buf, cidx)
    s01 = scale.reshape(T, K)
    s0col = jnp.broadcast_to(s01[:, 0:1], (T, 128))
    s1col = jnp.broadcast_to(s01[:, 1:2], (T, 128))
    return _combine(g0, g1, s0col, s1col)
